# baseline (speedup 1.0000x reference)
import functools
import math

import jax
import jax.numpy as jnp
from jax.experimental import pallas as pl
from jax.experimental.pallas import tpu as pltpu

D_MODEL = 4096
NUM_EXPERTS = 64
TOP_K = 8
CAP_FACTOR = 1.25

TOKEN_BLOCK = 1024
D_SPLIT = 2


def _router_kernel(x_ref, w_ref, w_out, i_out, lb_out, z_out, ovf_out,
                   logits_acc, acc_ref, *, num_tokens, grid_n):
    i = pl.program_id(0)
    j = pl.program_id(1)

    @pl.when(jnp.logical_and(i == 0, j == 0))
    def _init():
        acc_ref[...] = jnp.zeros_like(acc_ref)

    partial = jax.lax.dot_general(
        w_ref[...], x_ref[...],
        dimension_numbers=(((1,), (1,)), ((), ())),
        preferred_element_type=jnp.float32)

    @pl.when(j == 0)
    def _first():
        logits_acc[...] = partial

    @pl.when(j == D_SPLIT - 1)
    def _epilogue():
        logits = logits_acc[...] + partial
        tb = logits.shape[1]
        iota_s = jax.lax.broadcasted_iota(jnp.int32, (NUM_EXPERTS, tb), 0)

        work = logits
        vals = []
        idxs = []
        for _ in range(TOP_K):
            m = jnp.max(work, axis=0, keepdims=True)
            idx = jnp.min(jnp.where(work == m, iota_s, NUM_EXPERTS),
                          axis=0, keepdims=True)
            vals.append(m)
            idxs.append(idx)
            work = jnp.where(iota_s == idx, -jnp.inf, work)

        topv = jnp.concatenate(vals, axis=0)
        topi = jnp.concatenate(idxs, axis=0)

        e = jnp.exp(topv - topv[0:1, :])
        w_out[...] = (e / jnp.sum(e, axis=0, keepdims=True)).T
        i_out[...] = topi.T

        counts = jnp.sum(jnp.where(work == -jnp.inf, 1.0, 0.0),
                         axis=1, keepdims=True)

        p = jnp.exp(logits - topv[0:1, :])
        p = p / jnp.sum(p, axis=0, keepdims=True)
        probs_sum = jnp.sum(p, axis=1, keepdims=True)

        z_sum = jnp.sum(logits * logits, keepdims=True).reshape(1, 1)

        acc_ref[:, 0:1] += counts
        acc_ref[:, 1:2] += probs_sum
        acc_ref[0:1, 2:3] += z_sum

        @pl.when(i == grid_n - 1)
        def _finish():
            total_counts = acc_ref[:, 0:1]
            total_probs = acc_ref[:, 1:2]
            total_assign = num_tokens * TOP_K
            freq = total_counts / (total_assign + 1e-8)
            avg_probs = total_probs / num_tokens
            lb_out[...] = (jnp.sum(freq * avg_probs, keepdims=True)
                           * NUM_EXPERTS).reshape(1, 1)
            z_out[...] = acc_ref[0:1, 2:3] / (num_tokens * NUM_EXPERTS)
            cap = float(math.ceil(CAP_FACTOR * num_tokens * TOP_K
                                  / NUM_EXPERTS))
            ovf_out[...] = jnp.sum(jnp.maximum(total_counts - cap, 0.0),
                                   keepdims=True).reshape(1, 1)


def kernel(x, W):
    B, S, D = x.shape
    num_tokens = B * S
    x_flat = x.reshape(num_tokens, D)

    tb = TOKEN_BLOCK
    grid_n = num_tokens // tb
    dc = D // D_SPLIT

    body = functools.partial(_router_kernel, num_tokens=num_tokens,
                             grid_n=grid_n)

    out_shapes = (
        jax.ShapeDtypeStruct((num_tokens, TOP_K), jnp.float32),
        jax.ShapeDtypeStruct((num_tokens, TOP_K), jnp.int32),
        jax.ShapeDtypeStruct((1, 1), jnp.float32),
        jax.ShapeDtypeStruct((1, 1), jnp.float32),
        jax.ShapeDtypeStruct((1, 1), jnp.float32),
    )
    out_specs = (
        pl.BlockSpec((tb, TOP_K), lambda i, j: (i, 0)),
        pl.BlockSpec((tb, TOP_K), lambda i, j: (i, 0)),
        pl.BlockSpec((1, 1), lambda i, j: (0, 0)),
        pl.BlockSpec((1, 1), lambda i, j: (0, 0)),
        pl.BlockSpec((1, 1), lambda i, j: (0, 0)),
    )
    in_specs = (
        pl.BlockSpec((tb, dc), lambda i, j: (i, j)),
        pl.BlockSpec((NUM_EXPERTS, dc), lambda i, j: (0, j)),
    )

    weights, indices, lb, z, ovf = pl.pallas_call(
        body,
        grid=(grid_n, D_SPLIT),
        in_specs=in_specs,
        out_specs=out_specs,
        out_shape=out_shapes,
        scratch_shapes=[pltpu.VMEM((NUM_EXPERTS, tb), jnp.float32),
                        pltpu.VMEM((NUM_EXPERTS, 8), jnp.float32)],
    )(x_flat, W)

    return (weights.reshape(B, S, TOP_K),
            indices.reshape(B, S, TOP_K),
            lb[0, 0], z[0, 0], ovf[0, 0])


# speedup vs baseline: 1.1717x; 1.1717x over previous
import functools
import math

import jax
import jax.numpy as jnp
from jax.experimental import pallas as pl
from jax.experimental.pallas import tpu as pltpu

D_MODEL = 4096
NUM_EXPERTS = 64
TOP_K = 8
CAP_FACTOR = 1.25

TOKEN_BLOCK = 1024


def _router_kernel(x_ref, w_ref, w_out, i_out, lb_out, z_out, ovf_out,
                   acc_ref, *, num_tokens, grid_n):
    i = pl.program_id(0)

    @pl.when(i == 0)
    def _init():
        acc_ref[...] = jnp.zeros_like(acc_ref)

    logits = jax.lax.dot_general(
        w_ref[...], x_ref[...],
        dimension_numbers=(((1,), (1,)), ((), ())),
        preferred_element_type=jnp.float32)
    tb = logits.shape[1]
    iota_s = jax.lax.broadcasted_iota(jnp.int32, (NUM_EXPERTS, tb), 0)

    work = logits
    vals = []
    idxs = []
    for _ in range(TOP_K):
        m = jnp.max(work, axis=0, keepdims=True)
        idx = jnp.min(jnp.where(work == m, iota_s, NUM_EXPERTS),
                      axis=0, keepdims=True)
        vals.append(m)
        idxs.append(idx)
        work = jnp.where(iota_s == idx, -jnp.inf, work)

    topv = jnp.concatenate(vals, axis=0)
    topi = jnp.concatenate(idxs, axis=0)

    e = jnp.exp(topv - topv[0:1, :])
    w_out[...] = (e / jnp.sum(e, axis=0, keepdims=True)).T
    i_out[...] = topi.T

    counts = jnp.sum(jnp.where(work == -jnp.inf, 1.0, 0.0),
                     axis=1, keepdims=True)

    p = jnp.exp(logits - topv[0:1, :])
    p = p / jnp.sum(p, axis=0, keepdims=True)
    probs_sum = jnp.sum(p, axis=1, keepdims=True)

    z_sum = jnp.sum(logits * logits, keepdims=True).reshape(1, 1)

    acc_ref[:, 0:1] += counts
    acc_ref[:, 1:2] += probs_sum
    acc_ref[0:1, 2:3] += z_sum

    @pl.when(i == grid_n - 1)
    def _finish():
        total_counts = acc_ref[:, 0:1]
        total_probs = acc_ref[:, 1:2]
        total_assign = num_tokens * TOP_K
        freq = total_counts / (total_assign + 1e-8)
        avg_probs = total_probs / num_tokens
        lb_out[...] = (jnp.sum(freq * avg_probs, keepdims=True)
                       * NUM_EXPERTS).reshape(1, 1)
        z_out[...] = acc_ref[0:1, 2:3] / (num_tokens * NUM_EXPERTS)
        cap = float(math.ceil(CAP_FACTOR * num_tokens * TOP_K / NUM_EXPERTS))
        ovf_out[...] = jnp.sum(jnp.maximum(total_counts - cap, 0.0),
                               keepdims=True).reshape(1, 1)


def kernel(x, W):
    B, S, D = x.shape
    num_tokens = B * S
    x_flat = x.reshape(num_tokens, D)

    tb = TOKEN_BLOCK
    grid_n = num_tokens // tb

    body = functools.partial(_router_kernel, num_tokens=num_tokens,
                             grid_n=grid_n)

    out_shapes = (
        jax.ShapeDtypeStruct((num_tokens, TOP_K), jnp.float32),
        jax.ShapeDtypeStruct((num_tokens, TOP_K), jnp.int32),
        jax.ShapeDtypeStruct((1, 1), jnp.float32),
        jax.ShapeDtypeStruct((1, 1), jnp.float32),
        jax.ShapeDtypeStruct((1, 1), jnp.float32),
    )
    out_specs = (
        pl.BlockSpec((tb, TOP_K), lambda i: (i, 0)),
        pl.BlockSpec((tb, TOP_K), lambda i: (i, 0)),
        pl.BlockSpec((1, 1), lambda i: (0, 0)),
        pl.BlockSpec((1, 1), lambda i: (0, 0)),
        pl.BlockSpec((1, 1), lambda i: (0, 0)),
    )
    in_specs = (
        pl.BlockSpec((tb, D), lambda i: (i, 0)),
        pl.BlockSpec((NUM_EXPERTS, D), lambda i: (0, 0)),
    )

    weights, indices, lb, z, ovf = pl.pallas_call(
        body,
        grid=(grid_n,),
        in_specs=in_specs,
        out_specs=out_specs,
        out_shape=out_shapes,
        scratch_shapes=[pltpu.VMEM((NUM_EXPERTS, 8), jnp.float32)],
        compiler_params=pltpu.CompilerParams(
            vmem_limit_bytes=110 * 1024 * 1024),
    )(x_flat, W)

    return (weights.reshape(B, S, TOP_K),
            indices.reshape(B, S, TOP_K),
            lb[0, 0], z[0, 0], ovf[0, 0])
